# baseline (speedup 1.0000x reference)
import base64
import functools

import jax
import jax.numpy as jnp
import numpy as np
from jax import lax
from jax.experimental import pallas as pl
from jax.experimental.pallas import tpu as pltpu
from jax.experimental.pallas import tpu_sc as plsc

_B = 64
_S = 7
_CELLS = _B * _S * _S
_CH = 30
_NW = 16
_BPW = _B // _NW
_CPW = _CELLS // _NW
_FPW = _CPW * _CH
_GROUPS = (_CPW + 15) // 16
_NACC = 11
_ACCW = _NACC * 16
_NCW = 112

_COIN_B64 = (
    "1mx/t+uCj9LiwBtIkWbPYRELjGEGYPUi0/SpqtO3MW2I2mJ8smTKUo+DIXMdd166eIJcOzNX"
    "TdG9GAXHCZxvrLLRpf8sIxzkleMfiPhHMdBpLXMZgPeDiXj+8CLQsz3EEVF1+1/OqbA+uVzk"
    "GHVW2sV3pbDTb3xEshyFHhRbu6jDyotFyHwe2fM3ZlUnZSlG6Sf/7/bXVmiPApwViHmfvmsE"
    "LZrTCqE3Q2MRytpL/NyKlDvu2GwebpfqhiQgTdhW26hciWiCHyXEAPYx2wQ2Iu5erFjTGia0"
    "mQZicnWQuoYqfLuIqVM+JRnjA4q/jvH45Uos1gZgYdN4Jfsgqwpu1SsAMZ2GQdn4dfUVCntX"
    "ss9ncFKjuSgzdp1c+bZpOmNYM2Wqgb8bKtgglXRWw+WmTgfZerIunQzq33fr0DgDcuGetDmA"
    "HWU7EhEGqPyBdr/bAB4omh61OMw5pTKc/t9xwfIMaZDc3Wp0Bddn6sVu5sFc4PkfvNR7IoOj"
    "UeY8LOLUjBPdityAZJA="
)


@functools.cache
def _coin_words():
    bits = np.unpackbits(
        np.frombuffer(base64.b64decode(_COIN_B64), dtype=np.uint8))[:_CELLS]
    words = np.packbits(
        np.pad(bits, (0, _NCW * 32 - _CELLS)).reshape(-1, 32),
        axis=1, bitorder="little").view("<u4").reshape(-1)
    return [[int(w) - (1 << 32) if w >= (1 << 31) else int(w) for w in row]
            for row in words.astype(np.int64).reshape(7, 16)]


def _sc_body(pt_hbm, out0, out1, out2, out3,
             pred_v, targ_v, coinw_v, acc_v, shared_v, big_v, out_v):
    cid = lax.axis_index("c")
    sid = lax.axis_index("s")

    pltpu.sync_copy(pt_hbm.at[pl.ds(sid * _FPW, _FPW)], pred_v)
    pltpu.sync_copy(pt_hbm.at[pl.ds(_CELLS * _CH + sid * _FPW, _FPW)], targ_v)

    lane = lax.broadcasted_iota(jnp.int32, (16,), 0)
    zero = jnp.zeros((16,), jnp.float32)

    cw = _coin_words()
    for k in range(7):
        vec = jnp.full((16,), cw[k][0], jnp.int32)
        for l in range(1, 16):
            vec = jnp.where(lane == l, jnp.int32(cw[k][l]), vec)
        plsc.store_scatter(coinw_v, [lane + k * 16], vec)

    def group_body(g, carry):
        (n1, n2, nn, s1loc, s2loc, s1obj, s2obj, s1cls, s2cls, sn4, sn9) = carry
        cbase = g * 16 + lane
        valid = cbase < _CPW
        ccl = jnp.minimum(cbase, _CPW - 1)
        col0 = ccl * _CH

        def gat(ref, j):
            return plsc.load_gather(ref, [col0 + j])

        qloc1 = zero
        qloc2 = zero
        qcls = zero
        kept_p = {}
        kept_t = {}
        for j in range(_CH):
            p = gat(pred_v, j)
            t = gat(targ_v, j)
            if j == 4 or j == 9:
                kept_p[j] = p
                kept_t[j] = t
                continue
            d = p - t
            if j < 4:
                qloc1 = qloc1 + d * d
            elif j < 9:
                qloc2 = qloc2 + d * d
            else:
                qcls = qcls + d * d

        cg = sid * _CPW + ccl
        word = plsc.load_gather(coinw_v, [lax.div(cg, 32)])
        bit = lax.shift_right_logical(word, lax.rem(cg, 32)) & 1
        pick1 = bit > 0

        t4 = kept_t[4]
        t9 = kept_t[9]
        obj = t4 > 0.0
        t4p = jnp.where(obj & jnp.logical_not(pick1), 0.0, t4)
        t9p = jnp.where(obj & pick1, 0.0, t9)
        m1 = jnp.where((t4p > 0.0) & valid, 1.0, 0.0)
        m2 = jnp.where((t9p > 0.0) & valid, 1.0, 0.0)
        mn = jnp.where((t4p == 0.0) & (t9p == 0.0) & valid, 1.0, 0.0)
        d24 = (kept_p[4] - t4p) * (kept_p[4] - t4p)
        d29 = (kept_p[9] - t9p) * (kept_p[9] - t9p)

        return (n1 + m1, n2 + m2, nn + mn,
                s1loc + m1 * qloc1, s2loc + m2 * qloc2,
                s1obj + m1 * d24, s2obj + m2 * d29,
                s1cls + m1 * qcls, s2cls + m2 * qcls,
                sn4 + mn * d24, sn9 + mn * d29)

    init = tuple(zero for _ in range(_NACC))
    parts = plsc.parallel_loop(0, _GROUPS, unroll=4, carry=init)(group_body)

    for i in range(_NACC):
        plsc.store_scatter(acc_v, [lane + i * 16], parts[i])
    pltpu.sync_copy(acc_v, shared_v.at[pl.ds(sid * _ACCW, _ACCW)])
    plsc.subcore_barrier()

    @pl.when((cid == 0) & (sid == 0))
    def _():
        pltpu.sync_copy(shared_v, big_v)
        tot = [zero for _ in range(_NACC)]
        for w in range(_NW):
            for i in range(_NACC):
                tot[i] = tot[i] + plsc.load_gather(
                    big_v, [lane + (w * _NACC + i) * 16])
        n1, n2, nn, s1loc, s2loc, s1obj, s2obj, s1cls, s2cls, sn4, sn9 = (
            jnp.sum(t) for t in tot)

        def lanes(vals, default):
            out = jnp.full((16,), default, jnp.float32)
            for k, v in reversed(list(enumerate(vals))):
                out = jnp.where(lane == k, v, out)
            return out

        num1 = lanes([s1loc, s1cls, s1obj, sn4 + sn9], 0.0)
        den1 = lanes([4.0 * n1, 20.0 * n1, n1, nn], 1.0)
        num2 = lanes([s2loc, s2cls, s2obj], 0.0)
        den2 = lanes([4.0 * n2, 20.0 * n2, n2], 1.0)
        scale = lanes([5.0 / _B, 1.0 / _B, 1.0 / _B, 0.5 / _B], 0.0)
        res = (num1 / den1 + num2 / den2) * scale
        plsc.store_scatter(out_v, [lane * 16], res)
        pltpu.sync_copy(out_v.at[pl.ds(0, 1)], out0)
        pltpu.sync_copy(out_v.at[pl.ds(16, 1)], out1)
        pltpu.sync_copy(out_v.at[pl.ds(32, 1)], out2)
        pltpu.sync_copy(out_v.at[pl.ds(48, 1)], out3)


@functools.cache
def _build_sc_kernel():
    mesh = plsc.VectorSubcoreMesh(core_axis_name="c", subcore_axis_name="s")
    one = jax.ShapeDtypeStruct((1,), jnp.float32)
    return pl.kernel(
        _sc_body,
        mesh=mesh,
        compiler_params=pltpu.CompilerParams(needs_layout_passes=False),
        out_type=(one, one, one, one),
        scratch_types=[
            pltpu.VMEM((_FPW,), jnp.float32),
            pltpu.VMEM((_FPW,), jnp.float32),
            pltpu.VMEM((_NCW,), jnp.int32),
            pltpu.VMEM((_ACCW,), jnp.float32),
            pltpu.VMEM_SHARED((_NW * _ACCW,), jnp.float32),
            pltpu.VMEM((_NW * _ACCW,), jnp.float32),
            pltpu.VMEM((64,), jnp.float32),
        ],
    )


def kernel(pred, target):
    pt = jnp.concatenate([pred.reshape(-1), target.reshape(-1)])
    o0, o1, o2, o3 = _build_sc_kernel()(pt)
    return (o0.reshape(()), o1.reshape(()), o2.reshape(()), o3.reshape(()))


# speedup vs baseline: 1.1031x; 1.1031x over previous
import base64
import functools

import jax
import jax.numpy as jnp
import numpy as np
from jax import lax
from jax.experimental import pallas as pl
from jax.experimental.pallas import tpu as pltpu
from jax.experimental.pallas import tpu_sc as plsc

_B = 64
_S = 7
_CELLS = _B * _S * _S
_CH = 30
_NW = 16
_BPW = _B // _NW
_CPW = _CELLS // _NW
_FPW = _CPW * _CH
_GROUPS = (_CPW + 15) // 16
_NACC = 11
_ACCW = _NACC * 16
_NCW = 112

_COIN_B64 = (
    "1mx/t+uCj9LiwBtIkWbPYRELjGEGYPUi0/SpqtO3MW2I2mJ8smTKUo+DIXMdd166eIJcOzNX"
    "TdG9GAXHCZxvrLLRpf8sIxzkleMfiPhHMdBpLXMZgPeDiXj+8CLQsz3EEVF1+1/OqbA+uVzk"
    "GHVW2sV3pbDTb3xEshyFHhRbu6jDyotFyHwe2fM3ZlUnZSlG6Sf/7/bXVmiPApwViHmfvmsE"
    "LZrTCqE3Q2MRytpL/NyKlDvu2GwebpfqhiQgTdhW26hciWiCHyXEAPYx2wQ2Iu5erFjTGia0"
    "mQZicnWQuoYqfLuIqVM+JRnjA4q/jvH45Uos1gZgYdN4Jfsgqwpu1SsAMZ2GQdn4dfUVCntX"
    "ss9ncFKjuSgzdp1c+bZpOmNYM2Wqgb8bKtgglXRWw+WmTgfZerIunQzq33fr0DgDcuGetDmA"
    "HWU7EhEGqPyBdr/bAB4omh61OMw5pTKc/t9xwfIMaZDc3Wp0Bddn6sVu5sFc4PkfvNR7IoOj"
    "UeY8LOLUjBPdityAZJA="
)


@functools.cache
def _coin_words():
    bits = np.unpackbits(
        np.frombuffer(base64.b64decode(_COIN_B64), dtype=np.uint8))[:_CELLS]
    words = np.packbits(
        np.pad(bits, (0, _NCW * 32 - _CELLS)).reshape(-1, 32),
        axis=1, bitorder="little").view("<u4").reshape(-1)
    return [[int(w) - (1 << 32) if w >= (1 << 31) else int(w) for w in row]
            for row in words.astype(np.int64).reshape(7, 16)]


def _sc_body(pred_hbm, targ_hbm, out0, out1, out2, out3,
             pred_v, targ_v, coinw_v, acc_v, shared_v, big_v, out_v):
    cid = lax.axis_index("c")
    sid = lax.axis_index("s")

    pltpu.sync_copy(pred_hbm.at[pl.ds(sid * _FPW, _FPW)], pred_v)
    pltpu.sync_copy(targ_hbm.at[pl.ds(sid * _FPW, _FPW)], targ_v)

    lane = lax.broadcasted_iota(jnp.int32, (16,), 0)
    zero = jnp.zeros((16,), jnp.float32)

    cw = _coin_words()
    for k in range(7):
        vec = jnp.full((16,), cw[k][0], jnp.int32)
        for l in range(1, 16):
            vec = jnp.where(lane == l, jnp.int32(cw[k][l]), vec)
        plsc.store_scatter(coinw_v, [lane + k * 16], vec)

    def group_body(g, carry):
        (n1, n2, nn, s1loc, s2loc, s1obj, s2obj, s1cls, s2cls, sn4, sn9) = carry
        cbase = g * 16 + lane
        valid = cbase < _CPW
        ccl = jnp.minimum(cbase, _CPW - 1)
        col0 = ccl * _CH

        def gat(ref, j):
            return plsc.load_gather(ref, [col0 + j])

        qloc1 = zero
        qloc2 = zero
        qcls = zero
        kept_p = {}
        kept_t = {}
        for j in range(_CH):
            p = gat(pred_v, j)
            t = gat(targ_v, j)
            if j == 4 or j == 9:
                kept_p[j] = p
                kept_t[j] = t
                continue
            d = p - t
            if j < 4:
                qloc1 = qloc1 + d * d
            elif j < 9:
                qloc2 = qloc2 + d * d
            else:
                qcls = qcls + d * d

        cg = sid * _CPW + ccl
        word = plsc.load_gather(coinw_v, [lax.div(cg, 32)])
        bit = lax.shift_right_logical(word, lax.rem(cg, 32)) & 1
        pick1 = bit > 0

        t4 = kept_t[4]
        t9 = kept_t[9]
        obj = t4 > 0.0
        t4p = jnp.where(obj & jnp.logical_not(pick1), 0.0, t4)
        t9p = jnp.where(obj & pick1, 0.0, t9)
        m1 = jnp.where((t4p > 0.0) & valid, 1.0, 0.0)
        m2 = jnp.where((t9p > 0.0) & valid, 1.0, 0.0)
        mn = jnp.where((t4p == 0.0) & (t9p == 0.0) & valid, 1.0, 0.0)
        d24 = (kept_p[4] - t4p) * (kept_p[4] - t4p)
        d29 = (kept_p[9] - t9p) * (kept_p[9] - t9p)

        return (n1 + m1, n2 + m2, nn + mn,
                s1loc + m1 * qloc1, s2loc + m2 * qloc2,
                s1obj + m1 * d24, s2obj + m2 * d29,
                s1cls + m1 * qcls, s2cls + m2 * qcls,
                sn4 + mn * d24, sn9 + mn * d29)

    init = tuple(zero for _ in range(_NACC))
    parts = plsc.parallel_loop(0, _GROUPS, unroll=2, carry=init)(group_body)

    for i in range(_NACC):
        plsc.store_scatter(acc_v, [lane + i * 16], parts[i])
    pltpu.sync_copy(acc_v, shared_v.at[pl.ds(sid * _ACCW, _ACCW)])
    plsc.subcore_barrier()

    @pl.when((cid == 0) & (sid == 0))
    def _():
        pltpu.sync_copy(shared_v, big_v)
        tot = [zero for _ in range(_NACC)]
        for w in range(_NW):
            for i in range(_NACC):
                tot[i] = tot[i] + plsc.load_gather(
                    big_v, [lane + (w * _NACC + i) * 16])
        n1, n2, nn, s1loc, s2loc, s1obj, s2obj, s1cls, s2cls, sn4, sn9 = (
            jnp.sum(t) for t in tot)

        def lanes(vals, default):
            out = jnp.full((16,), default, jnp.float32)
            for k, v in reversed(list(enumerate(vals))):
                out = jnp.where(lane == k, v, out)
            return out

        num1 = lanes([s1loc, s1cls, s1obj, sn4 + sn9], 0.0)
        den1 = lanes([4.0 * n1, 20.0 * n1, n1, nn], 1.0)
        num2 = lanes([s2loc, s2cls, s2obj], 0.0)
        den2 = lanes([4.0 * n2, 20.0 * n2, n2], 1.0)
        scale = lanes([5.0 / _B, 1.0 / _B, 1.0 / _B, 0.5 / _B], 0.0)
        res = (num1 / den1 + num2 / den2) * scale
        plsc.store_scatter(out_v, [lane * 16], res)
        pltpu.sync_copy(out_v.at[pl.ds(0, 1)], out0)
        pltpu.sync_copy(out_v.at[pl.ds(16, 1)], out1)
        pltpu.sync_copy(out_v.at[pl.ds(32, 1)], out2)
        pltpu.sync_copy(out_v.at[pl.ds(48, 1)], out3)


@functools.cache
def _build_sc_kernel():
    mesh = plsc.VectorSubcoreMesh(core_axis_name="c", subcore_axis_name="s")
    one = jax.ShapeDtypeStruct((1,), jnp.float32)
    return pl.kernel(
        _sc_body,
        mesh=mesh,
        compiler_params=pltpu.CompilerParams(needs_layout_passes=False),
        out_type=(one, one, one, one),
        scratch_types=[
            pltpu.VMEM((_FPW,), jnp.float32),
            pltpu.VMEM((_FPW,), jnp.float32),
            pltpu.VMEM((_NCW,), jnp.int32),
            pltpu.VMEM((_ACCW,), jnp.float32),
            pltpu.VMEM_SHARED((_NW * _ACCW,), jnp.float32),
            pltpu.VMEM((_NW * _ACCW,), jnp.float32),
            pltpu.VMEM((64,), jnp.float32),
        ],
    )


def kernel(pred, target):
    o0, o1, o2, o3 = _build_sc_kernel()(pred.reshape(-1), target.reshape(-1))
    return (o0.reshape(()), o1.reshape(()), o2.reshape(()), o3.reshape(()))


# speedup vs baseline: 1.1868x; 1.0759x over previous
import base64
import functools

import jax
import jax.numpy as jnp
import numpy as np
from jax import lax
from jax.experimental import pallas as pl
from jax.experimental.pallas import tpu as pltpu
from jax.experimental.pallas import tpu_sc as plsc

_B = 64
_S = 7
_CELLS = _B * _S * _S
_CH = 30
_NW = 16
_BPW = _B // _NW
_CPW = _CELLS // _NW
_FPW = _CPW * _CH
_GROUPS = (_CPW + 15) // 16
_NACC = 11
_ACCW = _NACC * 16
_NCW = 112

_COIN_B64 = (
    "1mx/t+uCj9LiwBtIkWbPYRELjGEGYPUi0/SpqtO3MW2I2mJ8smTKUo+DIXMdd166eIJcOzNX"
    "TdG9GAXHCZxvrLLRpf8sIxzkleMfiPhHMdBpLXMZgPeDiXj+8CLQsz3EEVF1+1/OqbA+uVzk"
    "GHVW2sV3pbDTb3xEshyFHhRbu6jDyotFyHwe2fM3ZlUnZSlG6Sf/7/bXVmiPApwViHmfvmsE"
    "LZrTCqE3Q2MRytpL/NyKlDvu2GwebpfqhiQgTdhW26hciWiCHyXEAPYx2wQ2Iu5erFjTGia0"
    "mQZicnWQuoYqfLuIqVM+JRnjA4q/jvH45Uos1gZgYdN4Jfsgqwpu1SsAMZ2GQdn4dfUVCntX"
    "ss9ncFKjuSgzdp1c+bZpOmNYM2Wqgb8bKtgglXRWw+WmTgfZerIunQzq33fr0DgDcuGetDmA"
    "HWU7EhEGqPyBdr/bAB4omh61OMw5pTKc/t9xwfIMaZDc3Wp0Bddn6sVu5sFc4PkfvNR7IoOj"
    "UeY8LOLUjBPdityAZJA="
)


@functools.cache
def _coin_words():
    bits = np.unpackbits(
        np.frombuffer(base64.b64decode(_COIN_B64), dtype=np.uint8))[:_CELLS]
    words = np.packbits(
        np.pad(bits, (0, _NCW * 32 - _CELLS)).reshape(-1, 32),
        axis=1, bitorder="little").view("<u4").reshape(-1)
    return [[int(w) - (1 << 32) if w >= (1 << 31) else int(w) for w in row]
            for row in words.astype(np.int64).reshape(7, 16)]


def _sc_body(pred_hbm, targ_hbm, out0, out1, out2, out3,
             pred_v, targ_v, coinw_v, acc_v, shared_v, big_v, out_v):
    cid = lax.axis_index("c")
    sid = lax.axis_index("s")

    pltpu.sync_copy(pred_hbm.at[pl.ds(sid * _FPW, _FPW)], pred_v)
    pltpu.sync_copy(targ_hbm.at[pl.ds(sid * _FPW, _FPW)], targ_v)

    lane = lax.broadcasted_iota(jnp.int32, (16,), 0)
    zero = jnp.zeros((16,), jnp.float32)

    cw = _coin_words()
    for k in range(7):
        vec = jnp.full((16,), cw[k][0], jnp.int32)
        for l in range(1, 16):
            vec = jnp.where(lane == l, jnp.int32(cw[k][l]), vec)
        plsc.store_scatter(coinw_v, [lane + k * 16], vec)

    def group_body(g, carry):
        (n1, n2, nn, s1loc, s2loc, s1obj, s2obj, s1cls, s2cls, sn4, sn9) = carry
        cbase = g * 16 + lane
        valid = cbase < _CPW
        ccl = jnp.minimum(cbase, _CPW - 1)
        col0 = ccl * _CH

        def gat(ref, j):
            return plsc.load_gather(ref, [col0 + j])

        qloc1 = zero
        qloc2 = zero
        qcls = zero
        kept_p = {}
        kept_t = {}
        for j in range(_CH):
            p = gat(pred_v, j)
            t = gat(targ_v, j)
            if j == 4 or j == 9:
                kept_p[j] = p
                kept_t[j] = t
                continue
            d = p - t
            if j < 4:
                qloc1 = qloc1 + d * d
            elif j < 9:
                qloc2 = qloc2 + d * d
            else:
                qcls = qcls + d * d

        cg = sid * _CPW + ccl
        word = plsc.load_gather(coinw_v, [lax.div(cg, 32)])
        bit = lax.shift_right_logical(word, lax.rem(cg, 32)) & 1
        pick1 = bit > 0

        t4 = kept_t[4]
        t9 = kept_t[9]
        obj = t4 > 0.0
        t4p = jnp.where(obj & jnp.logical_not(pick1), 0.0, t4)
        t9p = jnp.where(obj & pick1, 0.0, t9)
        m1 = jnp.where((t4p > 0.0) & valid, 1.0, 0.0)
        m2 = jnp.where((t9p > 0.0) & valid, 1.0, 0.0)
        mn = jnp.where((t4p == 0.0) & (t9p == 0.0) & valid, 1.0, 0.0)
        d24 = (kept_p[4] - t4p) * (kept_p[4] - t4p)
        d29 = (kept_p[9] - t9p) * (kept_p[9] - t9p)

        return (n1 + m1, n2 + m2, nn + mn,
                s1loc + m1 * qloc1, s2loc + m2 * qloc2,
                s1obj + m1 * d24, s2obj + m2 * d29,
                s1cls + m1 * qcls, s2cls + m2 * qcls,
                sn4 + mn * d24, sn9 + mn * d29)

    init = tuple(zero for _ in range(_NACC))
    parts = plsc.parallel_loop(0, _GROUPS, unroll=2, carry=init)(group_body)

    for i in range(_NACC):
        plsc.store_scatter(acc_v, [lane + i * 16], parts[i])
    pltpu.sync_copy(acc_v, shared_v.at[pl.ds(sid * _ACCW, _ACCW)])
    plsc.subcore_barrier()

    @pl.when((cid == 0) & (sid == 0))
    def _():
        pltpu.sync_copy(shared_v, big_v)
        tot = [zero for _ in range(_NACC)]
        for w in range(_NW):
            for i in range(_NACC):
                tot[i] = tot[i] + plsc.load_gather(
                    big_v, [lane + (w * _NACC + i) * 16])
        n1, n2, nn, s1loc, s2loc, s1obj, s2obj, s1cls, s2cls, sn4, sn9 = (
            jnp.sum(t) for t in tot)

        def lanes(vals, default):
            out = jnp.full((16,), default, jnp.float32)
            for k, v in reversed(list(enumerate(vals))):
                out = jnp.where(lane == k, v, out)
            return out

        num1 = lanes([s1loc, s1cls, s1obj, sn4 + sn9], 0.0)
        den1 = lanes([4.0 * n1, 20.0 * n1, n1, nn], 1.0)
        num2 = lanes([s2loc, s2cls, s2obj], 0.0)
        den2 = lanes([4.0 * n2, 20.0 * n2, n2], 1.0)
        scale = lanes([5.0 / _B, 1.0 / _B, 1.0 / _B, 0.5 / _B], 0.0)
        res = (num1 / den1 + num2 / den2) * scale
        plsc.store_scatter(out_v, [lane * 16], res)
        pltpu.sync_copy(out_v.at[pl.ds(0, 1)], out0)
        pltpu.sync_copy(out_v.at[pl.ds(16, 1)], out1)
        pltpu.sync_copy(out_v.at[pl.ds(32, 1)], out2)
        pltpu.sync_copy(out_v.at[pl.ds(48, 1)], out3)


@functools.cache
def _build_sc_kernel():
    mesh = plsc.VectorSubcoreMesh(core_axis_name="c", subcore_axis_name="s",
                                  num_cores=1)
    one = jax.ShapeDtypeStruct((1,), jnp.float32)
    return pl.kernel(
        _sc_body,
        mesh=mesh,
        compiler_params=pltpu.CompilerParams(needs_layout_passes=False),
        out_type=(one, one, one, one),
        scratch_types=[
            pltpu.VMEM((_FPW,), jnp.float32),
            pltpu.VMEM((_FPW,), jnp.float32),
            pltpu.VMEM((_NCW,), jnp.int32),
            pltpu.VMEM((_ACCW,), jnp.float32),
            pltpu.VMEM_SHARED((_NW * _ACCW,), jnp.float32),
            pltpu.VMEM((_NW * _ACCW,), jnp.float32),
            pltpu.VMEM((64,), jnp.float32),
        ],
    )


def kernel(pred, target):
    o0, o1, o2, o3 = _build_sc_kernel()(pred.reshape(-1), target.reshape(-1))
    return (o0.reshape(()), o1.reshape(()), o2.reshape(()), o3.reshape(()))


# speedup vs baseline: 1.2489x; 1.0524x over previous
import base64
import functools

import jax
import jax.numpy as jnp
import numpy as np
from jax import lax
from jax.experimental import pallas as pl
from jax.experimental.pallas import tpu as pltpu
from jax.experimental.pallas import tpu_sc as plsc

_B = 64
_S = 7
_CELLS = _B * _S * _S
_CH = 30
_NW = 16
_BPW = _B // _NW
_CPW = _CELLS // _NW
_FPW = _CPW * _CH
_GROUPS = (_CPW + 15) // 16
_NACC = 11
_ACCW = _NACC * 16
_NCW = 112

_COIN_B64 = (
    "1mx/t+uCj9LiwBtIkWbPYRELjGEGYPUi0/SpqtO3MW2I2mJ8smTKUo+DIXMdd166eIJcOzNX"
    "TdG9GAXHCZxvrLLRpf8sIxzkleMfiPhHMdBpLXMZgPeDiXj+8CLQsz3EEVF1+1/OqbA+uVzk"
    "GHVW2sV3pbDTb3xEshyFHhRbu6jDyotFyHwe2fM3ZlUnZSlG6Sf/7/bXVmiPApwViHmfvmsE"
    "LZrTCqE3Q2MRytpL/NyKlDvu2GwebpfqhiQgTdhW26hciWiCHyXEAPYx2wQ2Iu5erFjTGia0"
    "mQZicnWQuoYqfLuIqVM+JRnjA4q/jvH45Uos1gZgYdN4Jfsgqwpu1SsAMZ2GQdn4dfUVCntX"
    "ss9ncFKjuSgzdp1c+bZpOmNYM2Wqgb8bKtgglXRWw+WmTgfZerIunQzq33fr0DgDcuGetDmA"
    "HWU7EhEGqPyBdr/bAB4omh61OMw5pTKc/t9xwfIMaZDc3Wp0Bddn6sVu5sFc4PkfvNR7IoOj"
    "UeY8LOLUjBPdityAZJA="
)


@functools.cache
def _coin_words():
    bits = np.unpackbits(
        np.frombuffer(base64.b64decode(_COIN_B64), dtype=np.uint8))[:_CELLS]
    words = np.packbits(
        np.pad(bits, (0, _NCW * 32 - _CELLS)).reshape(-1, 32),
        axis=1, bitorder="little").view("<u4").reshape(-1)
    return [[int(w) - (1 << 32) if w >= (1 << 31) else int(w) for w in row]
            for row in words.astype(np.int64).reshape(7, 16)]


def _sc_body(pred_hbm, targ_hbm, out0, out1, out2, out3,
             pred_v, targ_v, coinw_v, acc_v, shared_v, big_v, out_v,
             sem0, sem1):
    cid = lax.axis_index("c")
    sid = lax.axis_index("s")

    cp = pltpu.async_copy(pred_hbm.at[pl.ds(sid * _FPW, _FPW)], pred_v, sem0)
    ct = pltpu.async_copy(targ_hbm.at[pl.ds(sid * _FPW, _FPW)], targ_v, sem1)

    lane = lax.broadcasted_iota(jnp.int32, (16,), 0)
    zero = jnp.zeros((16,), jnp.float32)

    cw = _coin_words()
    for k in range(7):
        vec = jnp.full((16,), cw[k][0], jnp.int32)
        for l in range(1, 16):
            vec = jnp.where(lane == l, jnp.int32(cw[k][l]), vec)
        plsc.store_scatter(coinw_v, [lane + k * 16], vec)
    cp.wait()
    ct.wait()

    def group_body(g, carry):
        (n1, n2, nn, s1loc, s2loc, s1obj, s2obj, s1cls, s2cls, sn4, sn9) = carry
        cbase = g * 16 + lane
        valid = cbase < _CPW
        ccl = jnp.minimum(cbase, _CPW - 1)
        col0 = ccl * _CH

        def gat(ref, j):
            return plsc.load_gather(ref, [col0 + j])

        qloc1 = zero
        qloc2 = zero
        qcls = zero
        kept_p = {}
        kept_t = {}
        for j in range(_CH):
            p = gat(pred_v, j)
            t = gat(targ_v, j)
            if j == 4 or j == 9:
                kept_p[j] = p
                kept_t[j] = t
                continue
            d = p - t
            if j < 4:
                qloc1 = qloc1 + d * d
            elif j < 9:
                qloc2 = qloc2 + d * d
            else:
                qcls = qcls + d * d

        cg = sid * _CPW + ccl
        word = plsc.load_gather(coinw_v, [lax.div(cg, 32)])
        bit = lax.shift_right_logical(word, lax.rem(cg, 32)) & 1
        pick1 = bit > 0

        t4 = kept_t[4]
        t9 = kept_t[9]
        obj = t4 > 0.0
        t4p = jnp.where(obj & jnp.logical_not(pick1), 0.0, t4)
        t9p = jnp.where(obj & pick1, 0.0, t9)
        m1 = jnp.where((t4p > 0.0) & valid, 1.0, 0.0)
        m2 = jnp.where((t9p > 0.0) & valid, 1.0, 0.0)
        mn = jnp.where((t4p == 0.0) & (t9p == 0.0) & valid, 1.0, 0.0)
        d24 = (kept_p[4] - t4p) * (kept_p[4] - t4p)
        d29 = (kept_p[9] - t9p) * (kept_p[9] - t9p)

        return (n1 + m1, n2 + m2, nn + mn,
                s1loc + m1 * qloc1, s2loc + m2 * qloc2,
                s1obj + m1 * d24, s2obj + m2 * d29,
                s1cls + m1 * qcls, s2cls + m2 * qcls,
                sn4 + mn * d24, sn9 + mn * d29)

    init = tuple(zero for _ in range(_NACC))
    parts = plsc.parallel_loop(0, _GROUPS, unroll=2, carry=init)(group_body)

    mine = zero
    for i in range(_NACC):
        mine = jnp.where(lane == i, jnp.sum(parts[i]), mine)
    plsc.store_scatter(acc_v, [lane], mine)
    pltpu.sync_copy(acc_v, shared_v.at[pl.ds(sid * 16, 16)])
    plsc.subcore_barrier()

    @pl.when((cid == 0) & (sid == 0))
    def _():
        pltpu.sync_copy(shared_v, big_v)
        tot = zero
        for w in range(_NW):
            tot = tot + plsc.load_gather(big_v, [lane + w * 16])
        n1, n2, nn, s1loc, s2loc, s1obj, s2obj, s1cls, s2cls, sn4, sn9 = (
            jnp.sum(jnp.where(lane == i, tot, 0.0)) for i in range(_NACC))

        def lanes(vals, default):
            out = jnp.full((16,), default, jnp.float32)
            for k, v in reversed(list(enumerate(vals))):
                out = jnp.where(lane == k, v, out)
            return out

        num1 = lanes([s1loc, s1cls, s1obj, sn4 + sn9], 0.0)
        den1 = lanes([4.0 * n1, 20.0 * n1, n1, nn], 1.0)
        num2 = lanes([s2loc, s2cls, s2obj], 0.0)
        den2 = lanes([4.0 * n2, 20.0 * n2, n2], 1.0)
        scale = lanes([5.0 / _B, 1.0 / _B, 1.0 / _B, 0.5 / _B], 0.0)
        res = (num1 / den1 + num2 / den2) * scale
        plsc.store_scatter(out_v, [lane * 16], res)
        pltpu.sync_copy(out_v.at[pl.ds(0, 1)], out0)
        pltpu.sync_copy(out_v.at[pl.ds(16, 1)], out1)
        pltpu.sync_copy(out_v.at[pl.ds(32, 1)], out2)
        pltpu.sync_copy(out_v.at[pl.ds(48, 1)], out3)


@functools.cache
def _build_sc_kernel():
    mesh = plsc.VectorSubcoreMesh(core_axis_name="c", subcore_axis_name="s",
                                  num_cores=1)
    one = jax.ShapeDtypeStruct((1,), jnp.float32)
    return pl.kernel(
        _sc_body,
        mesh=mesh,
        compiler_params=pltpu.CompilerParams(needs_layout_passes=False),
        out_type=(one, one, one, one),
        scratch_types=[
            pltpu.VMEM((_FPW,), jnp.float32),
            pltpu.VMEM((_FPW,), jnp.float32),
            pltpu.VMEM((_NCW,), jnp.int32),
            pltpu.VMEM((16,), jnp.float32),
            pltpu.VMEM_SHARED((_NW * 16,), jnp.float32),
            pltpu.VMEM((_NW * 16,), jnp.float32),
            pltpu.VMEM((64,), jnp.float32),
            pltpu.SemaphoreType.DMA,
            pltpu.SemaphoreType.DMA,
        ],
    )


def kernel(pred, target):
    o0, o1, o2, o3 = _build_sc_kernel()(pred.reshape(-1), target.reshape(-1))
    return (o0.reshape(()), o1.reshape(()), o2.reshape(()), o3.reshape(()))
